# baseline (speedup 1.0000x reference)
import functools

import jax
import jax.numpy as jnp
from jax import lax
from jax.experimental import pallas as pl
from jax.experimental.pallas import tpu as pltpu
from jax.experimental.pallas import tpu_sc as plsc

N = 10000
D = 128
E = 320000
EPS = 1e-5

NC = 2
NS = 16
NW = NC * NS
EPW = E // NW
K = 64
NCH = 160
EPWP = NCH * K
NPAD = 10240
RPT = NPAD // NS

RB = 640
GRID = NPAD // RB


def _sc_degree(comb32):
    mesh = plsc.VectorSubcoreMesh(core_axis_name="c", subcore_axis_name="s")

    @functools.partial(
        pl.kernel,
        out_type=jax.ShapeDtypeStruct((NC, NPAD), jnp.float32),
        mesh=mesh,
        scratch_types=[
            pltpu.VMEM((NCH * K // 128, 128), jnp.int32),
            pltpu.VMEM((NCH, K), jnp.int32),
            pltpu.VMEM((K,), jnp.float32),
            pltpu.VMEM((RPT,), jnp.float32),
            pltpu.VMEM_SHARED((NPAD,), jnp.float32),
            pltpu.SemaphoreType.DMA,
        ],
    )
    def deg_kernel(comb_hbm, deg_hbm, idx_v, dst_v, ones_v, z_v, deg_sh, sem):
        c = lax.axis_index("c")
        s = lax.axis_index("s")
        wid = s * NC + c

        def zfill(i, _):
            z_v[pl.ds(i * 16, 16)] = jnp.zeros((16,), jnp.float32)
            return 0

        lax.fori_loop(0, RPT // 16, zfill, 0)
        pltpu.sync_copy(z_v, deg_sh.at[pl.ds(s * RPT, RPT)])

        def ofill(i, _):
            ones_v[pl.ds(i * 16, 16)] = jnp.ones((16,), jnp.float32)
            return 0

        lax.fori_loop(0, K // 16, ofill, 0)
        pltpu.sync_copy(comb_hbm.at[wid], idx_v)

        def unp(j, _):
            for m in range(K // 32):
                col = (j % 2) * K + K // 2 + 16 * m
                w = idx_v[j // 2, pl.ds(col, 16)]
                dst_v[j, pl.ds(32 * m, 16)] = w & 0xFFFF
                dst_v[j, pl.ds(32 * m + 16, 16)] = w >> 16
            return 0

        lax.fori_loop(0, NCH, unp, 0)
        plsc.subcore_barrier()

        def fire(j, _):
            pltpu.async_copy(ones_v, deg_sh.at[dst_v.at[j]], sem, add=True)
            return 0

        lax.fori_loop(0, NCH, fire, 0)

        def drain(j, _):
            pltpu.make_async_copy(
                ones_v, deg_sh.at[dst_v.at[j]], sem).wait()
            return 0

        lax.fori_loop(0, NCH, drain, 0)
        plsc.subcore_barrier()
        pltpu.sync_copy(deg_sh.at[pl.ds(s * RPT, RPT)],
                        deg_hbm.at[c, pl.ds(s * RPT, RPT)])

    return deg_kernel(comb32)


def _sc_scatter(hts, comb32):
    mesh = plsc.VectorSubcoreMesh(core_axis_name="c", subcore_axis_name="s")

    @functools.partial(
        pl.kernel,
        out_type=jax.ShapeDtypeStruct((NC, NPAD, D), jnp.float32),
        mesh=mesh,
        scratch_types=[
            pltpu.VMEM((NCH * K // 128, 128), jnp.int32),
            pltpu.VMEM((4, 2, K), jnp.int32),
            pltpu.VMEM((4, K, D), jnp.float32),
            pltpu.SemaphoreType.DMA,
            pltpu.SemaphoreType.DMA,
            pltpu.SemaphoreType.DMA,
            pltpu.SemaphoreType.DMA,
            pltpu.SemaphoreType.DMA,
            pltpu.SemaphoreType.DMA,
            pltpu.SemaphoreType.DMA,
            pltpu.SemaphoreType.DMA,
            pltpu.VMEM_SHARED((NPAD, D), jnp.float32),
        ],
    )
    def edge_kernel(hts_hbm, comb_hbm, agg_hbm, idx_v, st_v, buf_v,
                    sg0, sg1, sg2, sg3, ss0, ss1, ss2, ss3, acc_sh):
        c = lax.axis_index("c")
        s = lax.axis_index("s")
        wid = s * NC + c
        sgs = (sg0, sg1, sg2, sg3)
        sss = (ss0, ss1, ss2, ss3)

        def zfill(i, _):
            buf_v[0, i // 8, pl.ds((i % 8) * 16, 16)] = \
                jnp.zeros((16,), jnp.float32)
            return 0

        lax.fori_loop(0, K * (D // 16), zfill, 0)

        def zcopy(m, _):
            pltpu.sync_copy(buf_v.at[0],
                            acc_sh.at[pl.ds(s * RPT + m * K, K), :])
            return 0

        lax.fori_loop(0, RPT // K, zcopy, 0)
        assert RPT % K == 0
        pltpu.sync_copy(comb_hbm.at[wid], idx_v)
        plsc.subcore_barrier()

        def unpack(row, q):
            for side in (0, 1):
                for m in range(K // 32):
                    col = (q % 2) * K + side * (K // 2) + 16 * m
                    w = idx_v[row, pl.ds(col, 16)]
                    st_v[q, side, pl.ds(32 * m, 16)] = w & 0xFFFF
                    st_v[q, side, pl.ds(32 * m + 16, 16)] = w >> 16

        def g_issue(q):
            pltpu.async_copy(hts_hbm.at[st_v.at[q, 0]], buf_v.at[q], sgs[q])

        def g_wait(q):
            pltpu.make_async_copy(hts_hbm.at[st_v.at[q, 0]], buf_v.at[q],
                                  sgs[q]).wait()

        def s_issue(q):
            pltpu.async_copy(buf_v.at[q], acc_sh.at[st_v.at[q, 1]],
                             sss[q], add=True)

        def s_wait(q):
            pltpu.make_async_copy(buf_v.at[q], acc_sh.at[st_v.at[q, 1]],
                                  sss[q]).wait()

        for q in range(4):
            unpack(q // 2, q)
            g_issue(q)
        for q in range(4):
            g_wait(q)
            s_issue(q)

        def body(i, _):
            for q in range(4):
                s_wait(q)
                unpack(2 * i + q // 2, q)
                g_issue(q)
            for q in range(4):
                g_wait(q)
                s_issue(q)
            return 0

        lax.fori_loop(1, NCH // 4, body, 0)
        for q in range(4):
            s_wait(q)
        plsc.subcore_barrier()
        pltpu.sync_copy(acc_sh.at[pl.ds(s * RPT, RPT), :],
                        agg_hbm.at[c, pl.ds(s * RPT, RPT), :])

    return edge_kernel(hts, comb32)


def _tc_pre_body(x_r, degp_r, bg_r, bb_r, bm_r, bv_r, gw_r, gb_r, fw_r, fb_r,
                 hts_r, base_r, dinv_r):
    scale = bg_r[...] * lax.rsqrt(bv_r[...] + EPS)
    inp = jnp.maximum((x_r[...] - bm_r[...]) * scale + bb_r[...], 0.0)
    ht = jnp.dot(inp, gw_r[...], preferred_element_type=jnp.float32)
    deg = degp_r[0] + degp_r[1] + 1.0
    dinv = lax.rsqrt(deg)
    row = pl.program_id(0) * RB + lax.broadcasted_iota(jnp.int32, (RB, 1), 0)
    hts_r[...] = jnp.where(row < N, ht * dinv, 0.0)
    base_r[...] = inp + jnp.dot(inp, fw_r[...],
                                preferred_element_type=jnp.float32) \
        + fb_r[...] + gb_r[...]
    dinv_r[...] = dinv


def _tc_pre(x, degp3, bn_gamma, bn_beta, bn_mean, bn_var,
            gcn_W, gcn_b, fc1_W, fc1_b):
    vec = pl.BlockSpec((1, D), lambda j: (0, 0))
    mat = pl.BlockSpec((D, D), lambda j: (0, 0))
    return pl.pallas_call(
        _tc_pre_body,
        grid=(GRID,),
        in_specs=[
            pl.BlockSpec((RB, D), lambda j: (j, 0)),
            pl.BlockSpec((NC, RB, 1), lambda j: (0, j, 0)),
            vec, vec, vec, vec, mat, vec, mat, vec,
        ],
        out_specs=[
            pl.BlockSpec((RB, D), lambda j: (j, 0)),
            pl.BlockSpec((RB, D), lambda j: (j, 0)),
            pl.BlockSpec((RB, 1), lambda j: (j, 0)),
        ],
        out_shape=[
            jax.ShapeDtypeStruct((NPAD, D), jnp.float32),
            jax.ShapeDtypeStruct((NPAD, D), jnp.float32),
            jax.ShapeDtypeStruct((NPAD, 1), jnp.float32),
        ],
    )(x, degp3, bn_gamma, bn_beta, bn_mean, bn_var, gcn_W, gcn_b, fc1_W, fc1_b)


def _tc_post_body(aggp_r, hts_r, base_r, dinv_r, ow_r, ob_r, fin_r):
    agg = aggp_r[0] + aggp_r[1] + hts_r[...]
    node1 = dinv_r[...] * agg + base_r[...]
    fin_r[...] = jnp.dot(jnp.maximum(node1, 0.0), ow_r[...],
                         preferred_element_type=jnp.float32) + ob_r[...]


def _tc_post(aggp, hts, base, dinv, out_W, out_b):
    return pl.pallas_call(
        _tc_post_body,
        grid=(GRID,),
        in_specs=[
            pl.BlockSpec((NC, RB, D), lambda j: (0, j, 0)),
            pl.BlockSpec((RB, D), lambda j: (j, 0)),
            pl.BlockSpec((RB, D), lambda j: (j, 0)),
            pl.BlockSpec((RB, 1), lambda j: (j, 0)),
            pl.BlockSpec((D, D), lambda j: (0, 0)),
            pl.BlockSpec((1, D), lambda j: (0, 0)),
        ],
        out_specs=pl.BlockSpec((RB, D), lambda j: (j, 0)),
        out_shape=jax.ShapeDtypeStruct((N, D), jnp.float32),
    )(aggp, hts, base, dinv, out_W, out_b)


def kernel(x, edge_index, bn_gamma, bn_beta, bn_mean, bn_var,
           gcn_W, gcn_b, fc1_W, fc1_b, out_W, out_b):
    nd = NW * EPWP - E
    pad = N + jnp.arange(nd, dtype=jnp.int32) % (NPAD - N)
    epad = jnp.concatenate(
        [edge_index, jnp.broadcast_to(pad, (2, nd))], axis=1)
    g = epad.reshape(2, NW, NCH, K // 32, 2, 16)
    w = g[..., 0, :] | (g[..., 1, :] << 16)
    comb32 = jnp.stack([w[0], w[1]],
                       axis=2).reshape(NW, NCH * K // 128, 128)

    degp = _sc_degree(comb32)
    degp3 = degp.reshape(NC, NPAD, 1)

    hts, base, dinv = _tc_pre(
        x, degp3,
        bn_gamma.reshape(1, D), bn_beta.reshape(1, D),
        bn_mean.reshape(1, D), bn_var.reshape(1, D),
        gcn_W, gcn_b.reshape(1, D), fc1_W, fc1_b.reshape(1, D))

    aggp = _sc_scatter(hts, comb32)

    return _tc_post(aggp, hts, base, dinv, out_W, out_b.reshape(1, D))


# speedup vs baseline: 1.2467x; 1.2467x over previous
import functools

import jax
import jax.numpy as jnp
from jax import lax
from jax.experimental import pallas as pl
from jax.experimental.pallas import tpu as pltpu
from jax.experimental.pallas import tpu_sc as plsc

N = 10000
D = 128
E = 320000
EPS = 1e-5

NC = 2
NS = 16
NW = NC * NS
EPW = E // NW
K = 64
NCH = 160
EPWP = NCH * K
NPAD = 10240
RPT = NPAD // NS

RB = 640
GRID = NPAD // RB


def _sc_degree(comb):
    mesh = plsc.VectorSubcoreMesh(core_axis_name="c", subcore_axis_name="s")

    @functools.partial(
        pl.kernel,
        out_type=jax.ShapeDtypeStruct((NC, NPAD), jnp.float32),
        mesh=mesh,
        scratch_types=[
            pltpu.VMEM((NCH, 1, K), jnp.int32),
            pltpu.VMEM((K,), jnp.float32),
            pltpu.VMEM((RPT,), jnp.float32),
            pltpu.VMEM_SHARED((NPAD,), jnp.float32),
            pltpu.SemaphoreType.DMA,
        ],
    )
    def deg_kernel(comb_hbm, deg_hbm, idx_v, ones_v, z_v, deg_sh, sem):
        c = lax.axis_index("c")
        s = lax.axis_index("s")
        wid = s * NC + c

        def zfill(i, _):
            z_v[pl.ds(i * 16, 16)] = jnp.zeros((16,), jnp.float32)
            return 0

        lax.fori_loop(0, RPT // 16, zfill, 0)
        pltpu.sync_copy(z_v, deg_sh.at[pl.ds(s * RPT, RPT)])

        def ofill(i, _):
            ones_v[pl.ds(i * 16, 16)] = jnp.ones((16,), jnp.float32)
            return 0

        lax.fori_loop(0, K // 16, ofill, 0)
        pltpu.sync_copy(comb_hbm.at[wid], idx_v)
        plsc.subcore_barrier()

        def fire(j, _):
            pltpu.async_copy(ones_v, deg_sh.at[idx_v.at[j, 0]], sem, add=True)
            return 0

        lax.fori_loop(0, NCH, fire, 0)

        def drain(j, _):
            pltpu.make_async_copy(
                ones_v, deg_sh.at[idx_v.at[j, 0]], sem).wait()
            return 0

        lax.fori_loop(0, NCH, drain, 0)
        plsc.subcore_barrier()
        pltpu.sync_copy(deg_sh.at[pl.ds(s * RPT, RPT)],
                        deg_hbm.at[c, pl.ds(s * RPT, RPT)])

    return deg_kernel(comb)


def _sc_scatter(hts, comb32):
    mesh = plsc.VectorSubcoreMesh(core_axis_name="c", subcore_axis_name="s")

    @functools.partial(
        pl.kernel,
        out_type=jax.ShapeDtypeStruct((NC, NPAD, D), jnp.float32),
        mesh=mesh,
        scratch_types=[
            pltpu.VMEM((NCH * K // 128, 128), jnp.int32),
            pltpu.VMEM((4, 2, K), jnp.int32),
            pltpu.VMEM((4, K, D), jnp.float32),
            pltpu.SemaphoreType.DMA,
            pltpu.SemaphoreType.DMA,
            pltpu.SemaphoreType.DMA,
            pltpu.SemaphoreType.DMA,
            pltpu.SemaphoreType.DMA,
            pltpu.SemaphoreType.DMA,
            pltpu.SemaphoreType.DMA,
            pltpu.SemaphoreType.DMA,
            pltpu.VMEM_SHARED((NPAD, D), jnp.float32),
        ],
    )
    def edge_kernel(hts_hbm, comb_hbm, agg_hbm, idx_v, st_v, buf_v,
                    sg0, sg1, sg2, sg3, ss0, ss1, ss2, ss3, acc_sh):
        c = lax.axis_index("c")
        s = lax.axis_index("s")
        wid = s * NC + c
        sgs = (sg0, sg1, sg2, sg3)
        sss = (ss0, ss1, ss2, ss3)

        def zfill(i, _):
            buf_v[0, i // 8, pl.ds((i % 8) * 16, 16)] = \
                jnp.zeros((16,), jnp.float32)
            return 0

        lax.fori_loop(0, K * (D // 16), zfill, 0)

        def zcopy(m, _):
            pltpu.sync_copy(buf_v.at[0],
                            acc_sh.at[pl.ds(s * RPT + m * K, K), :])
            return 0

        lax.fori_loop(0, RPT // K, zcopy, 0)
        assert RPT % K == 0
        pltpu.sync_copy(comb_hbm.at[wid], idx_v)
        plsc.subcore_barrier()

        def unpack(row, q):
            for side in (0, 1):
                for m in range(K // 32):
                    col = (q % 2) * K + side * (K // 2) + 16 * m
                    w = idx_v[row, pl.ds(col, 16)]
                    st_v[q, side, pl.ds(32 * m, 16)] = w & 0xFFFF
                    st_v[q, side, pl.ds(32 * m + 16, 16)] = w >> 16

        def g_issue(q):
            pltpu.async_copy(hts_hbm.at[st_v.at[q, 0]], buf_v.at[q], sgs[q])

        def g_wait(q):
            pltpu.make_async_copy(hts_hbm.at[st_v.at[q, 0]], buf_v.at[q],
                                  sgs[q]).wait()

        def s_issue(q):
            pltpu.async_copy(buf_v.at[q], acc_sh.at[st_v.at[q, 1]],
                             sss[q], add=True)

        def s_wait(q):
            pltpu.make_async_copy(buf_v.at[q], acc_sh.at[st_v.at[q, 1]],
                                  sss[q]).wait()

        for q in range(4):
            unpack(q // 2, q)
            g_issue(q)
        for q in range(4):
            g_wait(q)
            s_issue(q)

        def body(i, _):
            for q in range(4):
                s_wait(q)
                unpack(2 * i + q // 2, q)
                g_issue(q)
            for q in range(4):
                g_wait(q)
                s_issue(q)
            return 0

        lax.fori_loop(1, NCH // 4, body, 0)
        for q in range(4):
            s_wait(q)
        plsc.subcore_barrier()
        pltpu.sync_copy(acc_sh.at[pl.ds(s * RPT, RPT), :],
                        agg_hbm.at[c, pl.ds(s * RPT, RPT), :])

    return edge_kernel(hts, comb32)


def _tc_pre_body(x_r, degp_r, bg_r, bb_r, bm_r, bv_r, gw_r, gb_r, fw_r, fb_r,
                 hts_r, base_r, dinv_r):
    scale = bg_r[...] * lax.rsqrt(bv_r[...] + EPS)
    inp = jnp.maximum((x_r[...] - bm_r[...]) * scale + bb_r[...], 0.0)
    ht = jnp.dot(inp, gw_r[...], preferred_element_type=jnp.float32)
    deg = degp_r[0] + degp_r[1] + 1.0
    dinv = lax.rsqrt(deg)
    row = pl.program_id(0) * RB + lax.broadcasted_iota(jnp.int32, (RB, 1), 0)
    hts_r[...] = jnp.where(row < N, ht * dinv, 0.0)
    base_r[...] = inp + jnp.dot(inp, fw_r[...],
                                preferred_element_type=jnp.float32) \
        + fb_r[...] + gb_r[...]
    dinv_r[...] = dinv


def _tc_pre(x, degp3, bn_gamma, bn_beta, bn_mean, bn_var,
            gcn_W, gcn_b, fc1_W, fc1_b):
    vec = pl.BlockSpec((1, D), lambda j: (0, 0))
    mat = pl.BlockSpec((D, D), lambda j: (0, 0))
    return pl.pallas_call(
        _tc_pre_body,
        grid=(GRID,),
        in_specs=[
            pl.BlockSpec((RB, D), lambda j: (j, 0)),
            pl.BlockSpec((NC, RB, 1), lambda j: (0, j, 0)),
            vec, vec, vec, vec, mat, vec, mat, vec,
        ],
        out_specs=[
            pl.BlockSpec((RB, D), lambda j: (j, 0)),
            pl.BlockSpec((RB, D), lambda j: (j, 0)),
            pl.BlockSpec((RB, 1), lambda j: (j, 0)),
        ],
        out_shape=[
            jax.ShapeDtypeStruct((NPAD, D), jnp.float32),
            jax.ShapeDtypeStruct((NPAD, D), jnp.float32),
            jax.ShapeDtypeStruct((NPAD, 1), jnp.float32),
        ],
    )(x, degp3, bn_gamma, bn_beta, bn_mean, bn_var, gcn_W, gcn_b, fc1_W, fc1_b)


def _tc_post_body(aggp_r, hts_r, base_r, dinv_r, ow_r, ob_r, fin_r):
    agg = aggp_r[0] + aggp_r[1] + hts_r[...]
    node1 = dinv_r[...] * agg + base_r[...]
    fin_r[...] = jnp.dot(jnp.maximum(node1, 0.0), ow_r[...],
                         preferred_element_type=jnp.float32) + ob_r[...]


def _tc_post(aggp, hts, base, dinv, out_W, out_b):
    return pl.pallas_call(
        _tc_post_body,
        grid=(GRID,),
        in_specs=[
            pl.BlockSpec((NC, RB, D), lambda j: (0, j, 0)),
            pl.BlockSpec((RB, D), lambda j: (j, 0)),
            pl.BlockSpec((RB, D), lambda j: (j, 0)),
            pl.BlockSpec((RB, 1), lambda j: (j, 0)),
            pl.BlockSpec((D, D), lambda j: (0, 0)),
            pl.BlockSpec((1, D), lambda j: (0, 0)),
        ],
        out_specs=pl.BlockSpec((RB, D), lambda j: (j, 0)),
        out_shape=jax.ShapeDtypeStruct((N, D), jnp.float32),
    )(aggp, hts, base, dinv, out_W, out_b)


def kernel(x, edge_index, bn_gamma, bn_beta, bn_mean, bn_var,
           gcn_W, gcn_b, fc1_W, fc1_b, out_W, out_b):
    nd = EPWP - EPW
    pad = (N + (jnp.arange(NW, dtype=jnp.int32)[:, None] * 7
                + jnp.arange(nd, dtype=jnp.int32)[None, :]) % (NPAD - N))
    srcp = jnp.concatenate([edge_index[0].reshape(NW, EPW), pad], axis=1)
    dstp = jnp.concatenate([edge_index[1].reshape(NW, EPW), pad], axis=1)
    dstc = dstp.reshape(NW, NCH, 1, K)

    def pack16(a):
        g = a.reshape(NW, NCH, K // 32, 2, 16)
        return (g[..., 0, :] | (g[..., 1, :] << 16)).reshape(NW, NCH, K // 2)

    comb32 = jnp.stack([pack16(srcp), pack16(dstp)],
                       axis=2).reshape(NW, NCH * K // 128, 128)

    degp = _sc_degree(dstc)
    degp3 = degp.reshape(NC, NPAD, 1)

    hts, base, dinv = _tc_pre(
        x, degp3,
        bn_gamma.reshape(1, D), bn_beta.reshape(1, D),
        bn_mean.reshape(1, D), bn_var.reshape(1, D),
        gcn_W, gcn_b.reshape(1, D), fc1_W, fc1_b.reshape(1, D))

    aggp = _sc_scatter(hts, comb32)

    return _tc_post(aggp, hts, base, dinv, out_W, out_b.reshape(1, D))
